# baseline (speedup 1.0000x reference)
import functools

import jax
import jax.numpy as jnp
from jax import lax
from jax.experimental import pallas as pl
from jax.experimental.pallas import tpu as pltpu
from jax.experimental.pallas import tpu_sc as plsc

NC = 2
NS = 16
NW = NC * NS
LANES = 16
CHUNK = 128


def _gather_scale(idx2d, table):
    nchunks_total, chunk = idx2d.shape
    _, d = table.shape
    nch = nchunks_total // NW
    total = nchunks_total * chunk
    scale = jnp.float32(d) ** 0.5

    mesh = plsc.VectorSubcoreMesh(core_axis_name="c", subcore_axis_name="s")

    @functools.partial(
        pl.kernel,
        mesh=mesh,
        compiler_params=pltpu.CompilerParams(use_tc_tiling_on_sc=False),
        out_type=jax.ShapeDtypeStruct((total, d), jnp.float32),
        scratch_types=[
            pltpu.VMEM((nch, chunk), jnp.int32),
            pltpu.VMEM((2, chunk, d), jnp.float32),
            pltpu.SemaphoreType.DMA,
            pltpu.SemaphoreType.DMA,
        ],
    )
    def k(idx_hbm, tab_hbm, out_hbm, idx_v, rows_v, g0, g1):
        wid = lax.axis_index("s") * NC + lax.axis_index("c")
        cbase = wid * nch

        pltpu.sync_copy(idx_hbm.at[pl.ds(cbase, nch)], idx_v)

        def g_desc(j, b, sem):
            return pltpu.make_async_copy(
                tab_hbm.at[idx_v.at[j]], rows_v.at[b], sem)

        g_desc(0, 0, g0).start()
        g_desc(1, 1, g1).start()

        def pair(i, _):
            for b, sem in ((0, g0), (1, g1)):
                j = i * 2 + b
                g_desc(j, b, sem).wait()

                def scale_body(r, _, b=b):
                    for dr in range(4):
                        for c in range(d // LANES):
                            sl = (b, r * 4 + dr, pl.ds(c * LANES, LANES))
                            rows_v[sl] = rows_v[sl] * scale
                    return 0

                lax.fori_loop(0, chunk // 4, scale_body, 0)

                pltpu.sync_copy(
                    rows_v.at[b],
                    out_hbm.at[pl.ds((cbase + j) * chunk, chunk)])

                @pl.when(j + 2 < nch)
                def _(j=j, b=b, sem=sem):
                    g_desc(j + 2, b, sem).start()
            return 0

        lax.fori_loop(0, nch // 2, pair, 0)

    return k(idx2d, table)


def kernel(x, input_embedding):
    bx, hx = x.shape
    _, d = input_embedding.shape
    total = bx * hx
    idx2d = x.reshape(total // CHUNK, CHUNK)
    out = _gather_scale(idx2d, input_embedding)
    return out.reshape(bx, hx, d)
